# baseline (speedup 1.0000x reference)
import jax
import jax.numpy as jnp
from jax.experimental import pallas as pl
from jax.experimental.pallas import tpu as pltpu

_F32 = jnp.float32


def _dot(a, b, ca, cb):
    return jax.lax.dot_general(a, b, (((ca,), (cb,)), ((), ())),
                               preferred_element_type=_F32)


def _mm(a, b):
    return _dot(a, b, 1, 0)


def _mmT(a, b):
    return _dot(a, b, 1, 1)


def _outer(u, v):
    return _dot(u, v, 1, 1)


def _body(i_ref, a_ref, b_ref, c_ref,
          t1r_ref, t1c_ref, t2rr_ref, t2rc_ref, t2cr_ref,
          t3rr_ref, t3rc_ref, t3cr_ref, t4rr_ref, t4rc_ref, t4cr_ref,
          t6r_ref, t6c_ref, t7_ref, w8_ref, b8_ref, out_ref, f_ref):
    relu = lambda x: jnp.maximum(x, 0.0)
    m, n = a_ref.shape
    p = t2rr_ref.shape[0]
    z = i_ref[0]

    A0 = a_ref[...]
    brow = b_ref[...]
    crow = c_ref[...]

    rows = jax.lax.broadcasted_iota(jnp.int32, (p, p), 0)
    cols = jax.lax.broadcasted_iota(jnp.int32, (p, p), 1)
    ident = (rows == cols).astype(_F32)

    bcol = _mmT(ident, brow)
    ccol = _mmT(ident[:, :n], crow)

    f_ref[...] = jnp.zeros((m, p), _F32)
    f_ref[:, :n] = A0
    f_ref[:, n:n + 1] = bcol
    F = f_ref[...]

    rs = jnp.sum(F * F, axis=1, keepdims=True)
    ri = jax.lax.rsqrt(rs)
    rc = jax.lax.rsqrt(jnp.sum(crow * crow))

    G = _mmT(F, F)
    rp = ri * _mm(relu(G), ri)
    rn = ri * _mm(relu(-G), ri)
    wrc = ri * _mmT(A0, crow) * rc

    oh = (jax.lax.broadcasted_iota(jnp.int32, (1, m), 1) == z).astype(_F32)
    Fz = _mm(oh, F)
    riz = _mm(oh, ri)

    term1_r = riz * _mmT(t1r_ref[...], Fz[:, :n + 1])
    term1_c = riz * _mmT(t1c_ref[...], Fz[:, :n + 1])

    t4rr, t4rc, t4cr = t4rr_ref[...], t4rc_ref[...], t4cr_ref[...]

    u1 = _mm(t3rr_ref[...], relu(t4rr))
    u2 = _mm(t3rr_ref[...], relu(-t4rr))
    v1 = _mm(t3rc_ref[...], relu(t4rc))
    v2 = _mm(t3rc_ref[...], relu(-t4rc))
    term3_r = (_outer(u1, rp - 1.0) + _outer(u2, rn)
               + _outer(v1, relu(wrc)) + _outer(v2, relu(-wrc)))

    srp = jnp.sum(relu(wrc))
    srn = jnp.sum(relu(-wrc))
    term3_c = _mm(t3cr_ref[...], relu(t4cr) * srp + relu(-t4cr) * srn)

    mu_r = _dot(F, ident, 0, 0)
    mu_c = ccol

    t2rr, t2rc, t2cr = t2rr_ref[...], t2rc_ref[...], t2cr_ref[...]
    cr = term1_r + term3_r
    rowsum = mu_c
    for _ in range(4):
        s = _mm(t2rc, mu_c)
        mu_r = relu(cr + _mm(t2rr, mu_r) + s)
        rowsum = jnp.sum(mu_r, axis=1, keepdims=True)
        mu_c = relu(term1_c + _mm(t2cr, rowsum) + term3_c)

    term6 = _mm(t6r_ref[...], rowsum) + _mm(t6c_ref[...], mu_c)
    muz = _mmT(mu_r, oh)
    term7 = _mm(t7_ref[...], muz)

    sig6 = jax.nn.sigmoid(term6)
    sig7 = jax.nn.sigmoid(term7)
    out_ref[...] = (_dot(sig6, w8_ref[:, :p], 0, 1)
                    + _dot(sig7, w8_ref[:, p:], 0, 1)
                    + b8_ref[...])


def kernel(A, b, c, i, theta1r, theta1c, theta2rr, theta2rc, theta2cr,
           theta3rr, theta3rc, theta3cr, theta4rr, theta4rc, theta4cr,
           theta6r, theta6c, theta7, W8, b8):
    m, n = A.shape[1], A.shape[2]
    p = theta2rr.shape[0]
    vmem = pl.BlockSpec(memory_space=pltpu.VMEM)
    return pl.pallas_call(
        _body,
        out_shape=jax.ShapeDtypeStruct((1, 2), _F32),
        in_specs=[pl.BlockSpec(memory_space=pltpu.SMEM)] + [vmem] * 19,
        out_specs=vmem,
        scratch_shapes=[pltpu.VMEM((m, p), _F32)],
    )(i, A[0], b, c,
      theta1r, theta1c, theta2rr, theta2rc, theta2cr,
      theta3rr, theta3rc, theta3cr, theta4rr, theta4rc, theta4cr,
      theta6r, theta6c, theta7, W8, b8.reshape(1, 2))


# speedup vs baseline: 2.0482x; 2.0482x over previous
import jax
import jax.numpy as jnp
from jax.experimental import pallas as pl
from jax.experimental.pallas import tpu as pltpu

_F32 = jnp.float32


def _dot(a, b, ca, cb):
    return jax.lax.dot_general(a, b, (((ca,), (cb,)), ((), ())),
                               preferred_element_type=_F32)


def _mm(a, b):
    return _dot(a, b, 1, 0)


def _mmT(a, b):
    return _dot(a, b, 1, 1)


def _body(i_ref, at_ref, b_ref, c_ref,
          t1r_ref, t1c_ref, t2rr_ref, t2rc_ref, t2cr_ref,
          t3rr_ref, t3rc_ref, t3cr_ref, t4rr_ref, t4rc_ref, t4cr_ref,
          t6r_ref, t6c_ref, t7_ref, w8_ref, b8_ref, out_ref, ft_ref):
    relu = lambda x: jnp.maximum(x, 0.0)
    n, m = at_ref.shape
    p = t2rr_ref.shape[0]
    z = i_ref[0]

    AT = at_ref[...]
    brow = b_ref[...]
    crow = c_ref[...]

    G = _dot(AT, AT, 0, 0) + _dot(brow, brow, 0, 0)
    rows = jax.lax.broadcasted_iota(jnp.int32, (p, p), 0)
    cols = jax.lax.broadcasted_iota(jnp.int32, (p, p), 1)
    ident = (rows == cols).astype(_F32)
    rs = jnp.sum(G * ident, axis=1, keepdims=True)
    ri = jax.lax.rsqrt(rs)
    rc = jax.lax.rsqrt(jnp.sum(crow * crow))

    rp = ri * _mm(relu(G), ri)
    rn = ri * _mm(relu(-G), ri)
    wrc = ri * _dot(AT, crow, 0, 1) * rc

    oh = (jax.lax.broadcasted_iota(jnp.int32, (1, m), 1) == z).astype(_F32)
    az = _mmT(AT, oh)
    bz = _mmT(brow, oh)
    riz = _mm(oh, ri)

    t1r, t1c = t1r_ref[...], t1c_ref[...]
    term1_r = riz * (_dot(t1r[:n, :], az, 0, 0) + _dot(t1r[n:, :], bz, 0, 0))
    term1_c = riz * (_dot(t1c[:n, :], az, 0, 0) + _dot(t1c[n:, :], bz, 0, 0))

    t4rr = t4rr_ref[...]
    t4rc = t4rc_ref[...]
    t4cr = t4cr_ref[...]

    u1 = _mmT(t3rr_ref[...], relu(t4rr))
    u2 = _mmT(t3rr_ref[...], relu(-t4rr))
    v1 = _mmT(t3rc_ref[...], relu(t4rc))
    v2 = _mmT(t3rc_ref[...], relu(-t4rc))
    UV = jnp.concatenate([u1, u2, v1, v2], axis=1)
    R4 = jnp.concatenate([rp - 1.0, rn, relu(wrc), relu(-wrc)], axis=1)
    term3_r = _mmT(UV, R4)

    srp = jnp.sum(relu(wrc))
    srn = jnp.sum(relu(-wrc))
    term3_c = _mmT(t3cr_ref[...], relu(t4cr) * srp + relu(-t4cr) * srn)

    ft_ref[...] = jnp.zeros((p, m), _F32)
    ft_ref[:n, :] = AT
    ft_ref[n:n + 1, :] = brow
    mu_r = ft_ref[...]
    mu_c = _mmT(ident[:, :n], crow)

    t2rr, t2rc, t2cr = t2rr_ref[...], t2rc_ref[...], t2cr_ref[...]
    cr = term1_r + term3_r
    rowsum = mu_c
    for _ in range(4):
        s = _mm(t2rc, mu_c)
        mu_r = relu(cr + _mm(t2rr, mu_r) + s)
        rowsum = jnp.sum(mu_r, axis=1, keepdims=True)
        mu_c = relu(term1_c + _mm(t2cr, rowsum) + term3_c)

    term6 = _mm(t6r_ref[...], rowsum) + _mm(t6c_ref[...], mu_c)
    muz = _mmT(mu_r, oh)
    term7 = _mm(t7_ref[...], muz)

    sig6 = jax.nn.sigmoid(term6)
    sig7 = jax.nn.sigmoid(term7)
    out_ref[...] = (_dot(sig6, w8_ref[:, :p], 0, 1)
                    + _dot(sig7, w8_ref[:, p:], 0, 1)
                    + b8_ref[...])


def kernel(A, b, c, i, theta1r, theta1c, theta2rr, theta2rc, theta2cr,
           theta3rr, theta3rc, theta3cr, theta4rr, theta4rc, theta4cr,
           theta6r, theta6c, theta7, W8, b8):
    m, n = A.shape[1], A.shape[2]
    p = theta2rr.shape[0]
    vmem = pl.BlockSpec(memory_space=pltpu.VMEM)
    return pl.pallas_call(
        _body,
        out_shape=jax.ShapeDtypeStruct((1, 2), _F32),
        in_specs=[pl.BlockSpec(memory_space=pltpu.SMEM)] + [vmem] * 19,
        out_specs=vmem,
        scratch_shapes=[pltpu.VMEM((p, m), _F32)],
    )(i, A[0].T, b, c,
      theta1r.T, theta1c.T, theta2rr, theta2rc, theta2cr,
      theta3rr, theta3rc, theta3cr, theta4rr.T, theta4rc.T, theta4cr.T,
      theta6r, theta6c, theta7, W8, b8.reshape(1, 2))
